# baseline (speedup 1.0000x reference)
import functools

import jax
import jax.numpy as jnp
from jax import lax
from jax.experimental import pallas as pl
from jax.experimental.pallas import tpu as pltpu
from jax.experimental.pallas import tpu_sc as plsc

LANES = 16
NC = 2
NS = 16
NW = NC * NS
NBUF = 2
DRING = 4
NSPLIT = 4
PHASE_FRAC = (0.3, 0.3, 0.3, 0.1)


def _matmul_bias_kernel(x_ref, w_ref, b_ref, o_ref):
    o_ref[...] = (
        jnp.dot(x_ref[...], w_ref[...], preferred_element_type=jnp.float32)
        + b_ref[...]
    )


def _matmul_kernel(x_ref, w_ref, o_ref):
    o_ref[...] = jnp.dot(x_ref[...], w_ref[...], preferred_element_type=jnp.float32)


def _head_kernel(n, a, nsplit, x_ref, *rest):
    agg_refs = rest[:nsplit]
    (wux_ref, wua_ref, bu_ref, wmu_ref, bmu_ref, sp_ref, mu_ref,
     sig_ref) = rest[nsplit:]
    agg = agg_refs[0][0, :n, :] + agg_refs[0][1, :n, :]
    for r in agg_refs[1:]:
        agg = agg + r[0, :n, :] + r[1, :n, :]
    h = jnp.maximum(
        jnp.dot(x_ref[...], wux_ref[...], preferred_element_type=jnp.float32)
        + jnp.dot(agg, wua_ref[...], preferred_element_type=jnp.float32)
        + bu_ref[...],
        0.0,
    )
    mu_ref[...] = (
        jnp.dot(h, wmu_ref[...], preferred_element_type=jnp.float32) + bmu_ref[...]
    )
    sig_ref[...] = jnp.broadcast_to(jnp.exp(sp_ref[...]), (n, a))


def _sc_edge_kernel(CH, chunk0, nchunks_call, aggrows, h, src_hbm, dst_hbm, p_hbm,
                    q_hbm, zero_hbm, out_hbm, agg_s, src_v, dst_v, p_v, q_v,
                    isem, gsem, ssem):
    c = lax.axis_index("c")
    s = lax.axis_index("s")
    wid = c * NS + s
    rows_per_tile = aggrows // NS

    base_cnt = nchunks_call // NW
    extra = nchunks_call % NW
    start_w = wid * base_cnt + jnp.minimum(wid, extra)
    cnt = base_cnt + jnp.where(wid < extra, 1, 0)

    pltpu.sync_copy(
        zero_hbm.at[pl.ds(s * rows_per_tile, rows_per_tile)],
        agg_s.at[pl.ds(s * rows_per_tile, rows_per_tile)],
    )
    plsc.subcore_barrier()

    def issue_idx(i, b):
        base = (chunk0 + start_w + i) * CH
        pltpu.async_copy(src_hbm.at[pl.ds(base, CH)], src_v[b], isem[b])
        for d in range(DRING):
            @pl.when(i % DRING == d)
            def _():
                pltpu.async_copy(dst_hbm.at[pl.ds(base, CH)], dst_v[d], isem[b])

    def wait_idx(b):
        pltpu.make_async_copy(src_hbm.at[pl.ds(0, CH)], src_v[b], isem[b]).wait()
        pltpu.make_async_copy(dst_hbm.at[pl.ds(0, CH)], dst_v[0], isem[b]).wait()

    def issue_data(i, b):
        qbase = (start_w + i) * CH
        pltpu.async_copy(p_hbm.at[src_v[b]], p_v[b], gsem[b])
        pltpu.async_copy(q_hbm.at[pl.ds(qbase, CH)], q_v[b], gsem[b])

    def wait_data(b):
        pltpu.make_async_copy(p_hbm.at[src_v[b]], p_v[b], gsem[b]).wait()
        pltpu.make_async_copy(q_hbm.at[pl.ds(0, CH)], q_v[b], gsem[b]).wait()

    def issue_scatter(i, b):
        for d in range(DRING):
            @pl.when(i % DRING == d)
            def _():
                pltpu.async_copy(p_v[b], agg_s.at[dst_v[d]], ssem[b], add=True)

    def wait_scatter(b):
        pltpu.make_async_copy(p_v[b], agg_s.at[dst_v[0]], ssem[b]).wait()

    issue_idx(0, 0)

    @pl.when(1 < cnt)
    def _():
        issue_idx(1, 1)

    wait_idx(0)
    issue_data(0, 0)

    def outer(g, carry):
        for b in range(NBUF):
            i = g * NBUF + b
            bn = (b + 1) % NBUF

            @pl.when(i < cnt)
            def _():
                @pl.when(i + 1 < cnt)
                def _():
                    @pl.when(i >= 1)
                    def _():
                        wait_scatter(bn)
                    wait_idx(bn)
                    issue_data(i + 1, bn)

                wait_data(b)

                @plsc.parallel_loop(0, CH, step=1, unroll=4)
                def _row(r):
                    for cc in range(h // LANES):
                        sl = pl.ds(cc * LANES, LANES)
                        p_v[b][r, sl] = jnp.maximum(
                            p_v[b][r, sl] + q_v[b][r, sl], 0.0
                        )

                issue_scatter(i, b)

                @pl.when(i + NBUF < cnt)
                def _():
                    issue_idx(i + NBUF, b)
        return carry

    lax.fori_loop(0, (cnt + NBUF - 1) // NBUF, outer, 0)
    wait_scatter(0)

    @pl.when(1 < cnt)
    def _():
        wait_scatter(1)

    plsc.subcore_barrier()

    pltpu.sync_copy(
        agg_s.at[pl.ds(s * rows_per_tile, rows_per_tile)],
        out_hbm.at[c, pl.ds(s * rows_per_tile, rows_per_tile)],
    )


def kernel(x, edge_index, edge_attr, W_msg, b_msg, W_upd, b_upd, W_mu, b_mu,
           sigma_param):
    n, d = x.shape
    e = edge_index.shape[1]
    de = edge_attr.shape[1]
    hdim = W_msg.shape[1]
    a = W_mu.shape[1]

    CH = 64
    for cand in (112, 96, 80, 64):
        if e % cand == 0:
            CH = cand
            break
    epad = ((e + CH - 1) // CH) * CH
    nchunks_total = epad // CH
    aggrows = ((n + NS * 8 - 1) // (NS * 8)) * (NS * 8)
    if aggrows <= n:
        aggrows += NS * 8

    src_p = edge_index[0]
    dst_p = edge_index[1]
    ea_p = edge_attr
    if epad != e:
        pad = epad - e
        src_p = jnp.concatenate([src_p, jnp.zeros((pad,), jnp.int32)])
        dst_p = jnp.concatenate([dst_p, jnp.full((pad,), n, jnp.int32)])
        ea_p = jnp.concatenate([ea_p, jnp.zeros((pad, de), jnp.float32)])

    w1 = W_msg[:d]
    w2 = W_msg[d:]
    wu_x = W_upd[:d]
    wu_a = W_upd[d:]

    p_arr = pl.pallas_call(
        _matmul_bias_kernel,
        out_shape=jax.ShapeDtypeStruct((n, hdim), jnp.float32),
    )(x, w1, b_msg.reshape(1, hdim))

    zeros_init = jnp.zeros((aggrows, hdim), jnp.float32)
    mesh = plsc.VectorSubcoreMesh(core_axis_name="c", subcore_axis_name="s")

    bounds = [0]
    for k in range(NSPLIT - 1):
        nxt = bounds[-1] + max(1, int(round(PHASE_FRAC[k] * nchunks_total)))
        bounds.append(min(nxt, nchunks_total - (NSPLIT - 1 - k)))
    bounds.append(nchunks_total)

    aggs = []
    for k in range(NSPLIT):
        c0, c1 = bounds[k], bounds[k + 1]
        rows = (c1 - c0) * CH
        ea_k = lax.slice_in_dim(ea_p, c0 * CH, c1 * CH, axis=0)
        be = CH
        for cand in range(8192, CH - 1, -8):
            if rows % cand == 0:
                be = cand
                break
        q_k = pl.pallas_call(
            _matmul_kernel,
            grid=(rows // be,),
            in_specs=[
                pl.BlockSpec((be, de), lambda i: (i, 0)),
                pl.BlockSpec((de, hdim), lambda i: (0, 0)),
            ],
            out_specs=pl.BlockSpec((be, hdim), lambda i: (i, 0)),
            out_shape=jax.ShapeDtypeStruct((rows, hdim), jnp.float32),
        )(ea_k, w2)

        agg_k = pl.kernel(
            functools.partial(_sc_edge_kernel, CH, c0, c1 - c0, aggrows, hdim),
            out_type=jax.ShapeDtypeStruct((NC, aggrows, hdim), jnp.float32),
            mesh=mesh,
            scratch_types=[
                pltpu.VMEM_SHARED((aggrows, hdim), jnp.float32),
                [pltpu.VMEM((CH,), jnp.int32) for _ in range(NBUF)],
                [pltpu.VMEM((CH,), jnp.int32) for _ in range(DRING)],
                [pltpu.VMEM((CH, hdim), jnp.float32) for _ in range(NBUF)],
                [pltpu.VMEM((CH, hdim), jnp.float32) for _ in range(NBUF)],
                [pltpu.SemaphoreType.DMA for _ in range(NBUF)],
                [pltpu.SemaphoreType.DMA for _ in range(NBUF)],
                [pltpu.SemaphoreType.DMA for _ in range(NBUF)],
            ],
        )(src_p, dst_p, p_arr, q_k, zeros_init)
        aggs.append(agg_k)

    mu, sigma = pl.pallas_call(
        functools.partial(_head_kernel, n, a, NSPLIT),
        out_shape=(
            jax.ShapeDtypeStruct((n, a), jnp.float32),
            jax.ShapeDtypeStruct((n, a), jnp.float32),
        ),
    )(x, *aggs, wu_x, wu_a, b_upd.reshape(1, hdim), W_mu, b_mu.reshape(1, a),
      sigma_param.reshape(1, a))

    return (mu, sigma)


# speedup vs baseline: 1.0543x; 1.0543x over previous
import functools

import jax
import jax.numpy as jnp
from jax import lax
from jax.experimental import pallas as pl
from jax.experimental.pallas import tpu as pltpu
from jax.experimental.pallas import tpu_sc as plsc

LANES = 16
NC = 2
NS = 16
NW = NC * NS
NBUF = 2
DRING = 4
NSPLIT = 2
PHASE_FRAC = (0.5, 0.5)


def _matmul_bias_kernel(x_ref, w_ref, b_ref, o_ref):
    o_ref[...] = (
        jnp.dot(x_ref[...], w_ref[...], preferred_element_type=jnp.float32)
        + b_ref[...]
    )


def _matmul_kernel(x_ref, w_ref, o_ref):
    o_ref[...] = jnp.dot(x_ref[...], w_ref[...], preferred_element_type=jnp.float32)


def _head_kernel(n, a, nsplit, x_ref, *rest):
    agg_refs = rest[:nsplit]
    (wux_ref, wua_ref, bu_ref, wmu_ref, bmu_ref, sp_ref, mu_ref,
     sig_ref) = rest[nsplit:]
    agg = agg_refs[0][0, :n, :] + agg_refs[0][1, :n, :]
    for r in agg_refs[1:]:
        agg = agg + r[0, :n, :] + r[1, :n, :]
    h = jnp.maximum(
        jnp.dot(x_ref[...], wux_ref[...], preferred_element_type=jnp.float32)
        + jnp.dot(agg, wua_ref[...], preferred_element_type=jnp.float32)
        + bu_ref[...],
        0.0,
    )
    mu_ref[...] = (
        jnp.dot(h, wmu_ref[...], preferred_element_type=jnp.float32) + bmu_ref[...]
    )
    sig_ref[...] = jnp.broadcast_to(jnp.exp(sp_ref[...]), (n, a))


def _sc_edge_kernel(CH, chunk0, nchunks_call, aggrows, h, src_hbm, dst_hbm, p_hbm,
                    q_hbm, zero_hbm, out_hbm, agg_s, src_v, dst_v, p_v, q_v,
                    isem, gsem, ssem):
    c = lax.axis_index("c")
    s = lax.axis_index("s")
    wid = c * NS + s
    rows_per_tile = aggrows // NS

    base_cnt = nchunks_call // NW
    extra = nchunks_call % NW
    start_w = wid * base_cnt + jnp.minimum(wid, extra)
    cnt = base_cnt + jnp.where(wid < extra, 1, 0)

    pltpu.sync_copy(
        zero_hbm.at[pl.ds(s * rows_per_tile, rows_per_tile)],
        agg_s.at[pl.ds(s * rows_per_tile, rows_per_tile)],
    )
    plsc.subcore_barrier()

    def issue_idx(i, b):
        base = (chunk0 + start_w + i) * CH
        pltpu.async_copy(src_hbm.at[pl.ds(base, CH)], src_v[b], isem[b])
        for d in range(DRING):
            @pl.when(i % DRING == d)
            def _():
                pltpu.async_copy(dst_hbm.at[pl.ds(base, CH)], dst_v[d], isem[b])

    def wait_idx(b):
        pltpu.make_async_copy(src_hbm.at[pl.ds(0, CH)], src_v[b], isem[b]).wait()
        pltpu.make_async_copy(dst_hbm.at[pl.ds(0, CH)], dst_v[0], isem[b]).wait()

    def issue_data(i, b):
        qbase = (start_w + i) * CH
        pltpu.async_copy(p_hbm.at[src_v[b]], p_v[b], gsem[b])
        pltpu.async_copy(q_hbm.at[pl.ds(qbase, CH)], q_v[b], gsem[b])

    def wait_data(b):
        pltpu.make_async_copy(p_hbm.at[src_v[b]], p_v[b], gsem[b]).wait()
        pltpu.make_async_copy(q_hbm.at[pl.ds(0, CH)], q_v[b], gsem[b]).wait()

    def issue_scatter(i, b):
        for d in range(DRING):
            @pl.when(i % DRING == d)
            def _():
                pltpu.async_copy(p_v[b], agg_s.at[dst_v[d]], ssem[b], add=True)

    def wait_scatter(b):
        pltpu.make_async_copy(p_v[b], agg_s.at[dst_v[0]], ssem[b]).wait()

    issue_idx(0, 0)

    @pl.when(1 < cnt)
    def _():
        issue_idx(1, 1)

    wait_idx(0)
    issue_data(0, 0)

    def outer(g, carry):
        for b in range(NBUF):
            i = g * NBUF + b
            bn = (b + 1) % NBUF

            @pl.when(i < cnt)
            def _():
                @pl.when(i + 1 < cnt)
                def _():
                    @pl.when(i >= 1)
                    def _():
                        wait_scatter(bn)
                    wait_idx(bn)
                    issue_data(i + 1, bn)

                wait_data(b)

                @plsc.parallel_loop(0, CH, step=1, unroll=4)
                def _row(r):
                    for cc in range(h // LANES):
                        sl = pl.ds(cc * LANES, LANES)
                        p_v[b][r, sl] = jnp.maximum(
                            p_v[b][r, sl] + q_v[b][r, sl], 0.0
                        )

                issue_scatter(i, b)

                @pl.when(i + NBUF < cnt)
                def _():
                    issue_idx(i + NBUF, b)
        return carry

    lax.fori_loop(0, (cnt + NBUF - 1) // NBUF, outer, 0)
    wait_scatter(0)

    @pl.when(1 < cnt)
    def _():
        wait_scatter(1)

    plsc.subcore_barrier()

    pltpu.sync_copy(
        agg_s.at[pl.ds(s * rows_per_tile, rows_per_tile)],
        out_hbm.at[c, pl.ds(s * rows_per_tile, rows_per_tile)],
    )


def kernel(x, edge_index, edge_attr, W_msg, b_msg, W_upd, b_upd, W_mu, b_mu,
           sigma_param):
    n, d = x.shape
    e = edge_index.shape[1]
    de = edge_attr.shape[1]
    hdim = W_msg.shape[1]
    a = W_mu.shape[1]

    CH = 64
    for cand in (112, 96, 80, 64):
        if e % cand == 0:
            CH = cand
            break
    epad = ((e + CH - 1) // CH) * CH
    nchunks_total = epad // CH
    aggrows = ((n + NS * 8 - 1) // (NS * 8)) * (NS * 8)
    if aggrows <= n:
        aggrows += NS * 8

    src_p = edge_index[0]
    dst_p = edge_index[1]
    ea_p = edge_attr
    if epad != e:
        pad = epad - e
        src_p = jnp.concatenate([src_p, jnp.zeros((pad,), jnp.int32)])
        dst_p = jnp.concatenate([dst_p, jnp.full((pad,), n, jnp.int32)])
        ea_p = jnp.concatenate([ea_p, jnp.zeros((pad, de), jnp.float32)])

    w1 = W_msg[:d]
    w2 = W_msg[d:]
    wu_x = W_upd[:d]
    wu_a = W_upd[d:]

    p_arr = pl.pallas_call(
        _matmul_bias_kernel,
        out_shape=jax.ShapeDtypeStruct((n, hdim), jnp.float32),
    )(x, w1, b_msg.reshape(1, hdim))

    zeros_init = jnp.zeros((aggrows, hdim), jnp.float32)
    mesh = plsc.VectorSubcoreMesh(core_axis_name="c", subcore_axis_name="s")

    bounds = [0]
    for k in range(NSPLIT - 1):
        nxt = bounds[-1] + max(1, int(round(PHASE_FRAC[k] * nchunks_total)))
        bounds.append(min(nxt, nchunks_total - (NSPLIT - 1 - k)))
    bounds.append(nchunks_total)

    aggs = []
    for k in range(NSPLIT):
        c0, c1 = bounds[k], bounds[k + 1]
        rows = (c1 - c0) * CH
        ea_k = lax.slice_in_dim(ea_p, c0 * CH, c1 * CH, axis=0)
        be = CH
        for cand in range(8192, CH - 1, -8):
            if rows % cand == 0:
                be = cand
                break
        q_k = pl.pallas_call(
            _matmul_kernel,
            grid=(rows // be,),
            in_specs=[
                pl.BlockSpec((be, de), lambda i: (i, 0)),
                pl.BlockSpec((de, hdim), lambda i: (0, 0)),
            ],
            out_specs=pl.BlockSpec((be, hdim), lambda i: (i, 0)),
            out_shape=jax.ShapeDtypeStruct((rows, hdim), jnp.float32),
        )(ea_k, w2)

        agg_k = pl.kernel(
            functools.partial(_sc_edge_kernel, CH, c0, c1 - c0, aggrows, hdim),
            out_type=jax.ShapeDtypeStruct((NC, aggrows, hdim), jnp.float32),
            mesh=mesh,
            scratch_types=[
                pltpu.VMEM_SHARED((aggrows, hdim), jnp.float32),
                [pltpu.VMEM((CH,), jnp.int32) for _ in range(NBUF)],
                [pltpu.VMEM((CH,), jnp.int32) for _ in range(DRING)],
                [pltpu.VMEM((CH, hdim), jnp.float32) for _ in range(NBUF)],
                [pltpu.VMEM((CH, hdim), jnp.float32) for _ in range(NBUF)],
                [pltpu.SemaphoreType.DMA for _ in range(NBUF)],
                [pltpu.SemaphoreType.DMA for _ in range(NBUF)],
                [pltpu.SemaphoreType.DMA for _ in range(NBUF)],
            ],
        )(src_p, dst_p, p_arr, q_k, zeros_init)
        aggs.append(agg_k)

    mu, sigma = pl.pallas_call(
        functools.partial(_head_kernel, n, a, NSPLIT),
        out_shape=(
            jax.ShapeDtypeStruct((n, a), jnp.float32),
            jax.ShapeDtypeStruct((n, a), jnp.float32),
        ),
    )(x, *aggs, wu_x, wu_a, b_upd.reshape(1, hdim), W_mu, b_mu.reshape(1, a),
      sigma_param.reshape(1, a))

    return (mu, sigma)
